# baseline (speedup 1.0000x reference)
import jax
import jax.numpy as jnp
from jax.experimental import pallas as pl
from jax.experimental.pallas import tpu as pltpu

NUM_EXPERTS = 4
CH = 96
HW = 224
BH = 56
BHH = 8
WPAD = 256
N_H = HW // BH


def _conv_kernel(gate_ref, w_ref, b_ref, xc_ref, xa_ref, xb_ref, out_ref):
    g = gate_ref[0]
    h = pl.program_id(1)

    xc = jnp.where(h == 0, 0.0, xc_ref[0])
    xb = jnp.where(h == N_H - 1, 0.0, xb_ref[0])
    slab = jnp.concatenate([xc, xa_ref[0], xb], axis=1)
    slab = jnp.pad(slab, ((0, 0), (0, 0), (1, WPAD - HW - 1)))
    slab2 = slab.reshape(CH, (BH + 2 * BHH) * WPAD).astype(jnp.bfloat16)

    accs = []
    for dw in range(3):
        acc = jnp.zeros((CH, BH * WPAD), dtype=jnp.float32)
        for dh in range(3):
            wm = jnp.zeros((CH, CH), dtype=jnp.float32)
            for e in range(NUM_EXPERTS):
                ge = g[0:1, e:e + 1]
                wm = wm + ge * w_ref[e, dh, dw]
            xs = slab2[:, (BHH - 1 + dh) * WPAD:
                          (BHH - 1 + dh) * WPAD + BH * WPAD]
            acc = acc + jnp.dot(wm.astype(jnp.bfloat16), xs,
                                preferred_element_type=jnp.float32)
        accs.append(acc.reshape(CH, BH, WPAD))

    bm = jnp.zeros((CH, 1), dtype=jnp.float32)
    for e in range(NUM_EXPERTS):
        bm = bm + g[0:1, e:e + 1] * b_ref[:, e:e + 1]

    out = (accs[0][:, :, 0:HW] + accs[1][:, :, 1:HW + 1]
           + accs[2][:, :, 2:HW + 2] + bm[:, :, None])
    out_ref[0] = jnp.maximum(out, 0.0)


def kernel(x, gate_values, W, b):
    B = x.shape[0]
    wt = jnp.transpose(W, (0, 3, 4, 1, 2))
    bt = jnp.transpose(b, (1, 0))
    gv = gate_values.reshape(B, 1, NUM_EXPERTS)
    n_halo = HW // BHH

    out = pl.pallas_call(
        _conv_kernel,
        grid=(B, N_H),
        in_specs=[
            pl.BlockSpec((1, 1, NUM_EXPERTS), lambda bb, h: (bb, 0, 0)),
            pl.BlockSpec((NUM_EXPERTS, 3, 3, CH, CH), lambda bb, h: (0, 0, 0, 0, 0)),
            pl.BlockSpec((CH, NUM_EXPERTS), lambda bb, h: (0, 0)),
            pl.BlockSpec((1, CH, BHH, HW),
                         lambda bb, h: (bb, 0, jnp.maximum((BH // BHH) * h - 1, 0), 0)),
            pl.BlockSpec((1, CH, BH, HW), lambda bb, h: (bb, 0, h, 0)),
            pl.BlockSpec((1, CH, BHH, HW),
                         lambda bb, h: (bb, 0, jnp.minimum((BH // BHH) * (h + 1), n_halo - 1), 0)),
        ],
        out_specs=pl.BlockSpec((1, CH, BH, HW), lambda bb, h: (bb, 0, h, 0)),
        out_shape=jax.ShapeDtypeStruct((B, CH, HW, HW), jnp.float32),
        compiler_params=pltpu.CompilerParams(
            dimension_semantics=("parallel", "arbitrary"),
        ),
    )(gv, wt, bt, x, x, x)
    return out


# speedup vs baseline: 1.0316x; 1.0316x over previous
import jax
import jax.numpy as jnp
from jax.experimental import pallas as pl
from jax.experimental.pallas import tpu as pltpu

NUM_EXPERTS = 4
CH = 96
HW = 224
BH = 56
BHH = 8
WPAD = 256
N_H = HW // BH


def _conv_kernel(gate_ref, w_ref, b_ref, xc_ref, xa_ref, xb_ref, out_ref):
    g = gate_ref[0]
    h = pl.program_id(1)

    xc = jnp.where(h == 0, 0.0, xc_ref[0]).astype(jnp.bfloat16)
    xb = jnp.where(h == N_H - 1, 0.0, xb_ref[0]).astype(jnp.bfloat16)
    slab = jnp.concatenate([xc, xa_ref[0].astype(jnp.bfloat16), xb], axis=1)
    slab = jnp.pad(slab, ((0, 0), (0, 0), (1, WPAD - HW - 1)))
    slab2 = slab.reshape(CH, (BH + 2 * BHH) * WPAD)

    accs = []
    for dw in range(3):
        acc = jnp.zeros((CH, BH * WPAD), dtype=jnp.float32)
        for dh in range(3):
            wm = jnp.zeros((CH, CH), dtype=jnp.float32)
            for e in range(NUM_EXPERTS):
                ge = g[0:1, e:e + 1]
                wm = wm + ge * w_ref[e, dh, dw]
            xs = slab2[:, (BHH - 1 + dh) * WPAD:
                          (BHH - 1 + dh) * WPAD + BH * WPAD]
            acc = acc + jnp.dot(wm.astype(jnp.bfloat16), xs,
                                preferred_element_type=jnp.float32)
        accs.append(acc.reshape(CH, BH, WPAD))

    bm = jnp.zeros((CH, 1), dtype=jnp.float32)
    for e in range(NUM_EXPERTS):
        bm = bm + g[0:1, e:e + 1] * b_ref[:, e:e + 1]

    out = (accs[0][:, :, 0:HW] + accs[1][:, :, 1:HW + 1]
           + accs[2][:, :, 2:HW + 2] + bm[:, :, None])
    out_ref[0] = jnp.maximum(out, 0.0)


def kernel(x, gate_values, W, b):
    B = x.shape[0]
    wt = jnp.transpose(W, (0, 3, 4, 1, 2))
    bt = jnp.transpose(b, (1, 0))
    gv = gate_values.reshape(B, 1, NUM_EXPERTS)
    n_halo = HW // BHH

    out = pl.pallas_call(
        _conv_kernel,
        grid=(B, N_H),
        in_specs=[
            pl.BlockSpec((1, 1, NUM_EXPERTS), lambda bb, h: (bb, 0, 0)),
            pl.BlockSpec((NUM_EXPERTS, 3, 3, CH, CH), lambda bb, h: (0, 0, 0, 0, 0)),
            pl.BlockSpec((CH, NUM_EXPERTS), lambda bb, h: (0, 0)),
            pl.BlockSpec((1, CH, BHH, HW),
                         lambda bb, h: (bb, 0, jnp.maximum((BH // BHH) * h - 1, 0), 0)),
            pl.BlockSpec((1, CH, BH, HW), lambda bb, h: (bb, 0, h, 0)),
            pl.BlockSpec((1, CH, BHH, HW),
                         lambda bb, h: (bb, 0, jnp.minimum((BH // BHH) * (h + 1), n_halo - 1), 0)),
        ],
        out_specs=pl.BlockSpec((1, CH, BH, HW), lambda bb, h: (bb, 0, h, 0)),
        out_shape=jax.ShapeDtypeStruct((B, CH, HW, HW), jnp.float32),
        compiler_params=pltpu.CompilerParams(
            dimension_semantics=("parallel", "arbitrary"),
        ),
    )(gv, wt, bt, x, x, x)
    return out
